# baseline (speedup 1.0000x reference)
import functools

import jax
import jax.numpy as jnp
from jax import lax
from jax.experimental import pallas as pl
from jax.experimental.pallas import tpu as pltpu
from jax.experimental.pallas import tpu_sc as plsc

B = 16384
K = 21
DIM = 10
V = 1000000

NW = 32
SPT = B // NW
CS = 128
NCHUNK = SPT // CS


def _build_kernel():
    mesh = plsc.VectorSubcoreMesh(core_axis_name="c", subcore_axis_name="s")

    @functools.partial(
        pl.kernel,
        mesh=mesh,
        compiler_params=pltpu.CompilerParams(needs_layout_passes=False,
                                             use_tc_tiling_on_sc=False),
        out_type=jax.ShapeDtypeStruct((K, B), jnp.float32),
        scratch_types=[
            pltpu.VMEM((NCHUNK, CS), jnp.int32),
            pltpu.VMEM((DIM * NCHUNK, CS), jnp.float32),
            pltpu.VMEM((K, CS), jnp.int32),
            pltpu.VMEM((DIM * K, CS), jnp.float32),
            pltpu.VMEM((K, SPT), jnp.float32),
            pltpu.SemaphoreType.DMA,
            pltpu.SemaphoreType.DMA,
        ],
    )
    def sc_kernel(idx_in_hbm, oidxT_hbm, win_hbm, woutT_hbm, out_hbm,
                  iidx_v, in_vals_v, oidx_v, vals_v, out_v,
                  sem_in, sem_out):
        wid = lax.axis_index("c") * 16 + lax.axis_index("s")
        base = wid * SPT

        for p in range(NCHUNK):
            pltpu.sync_copy(idx_in_hbm.at[pl.ds(base + p * CS, CS)],
                            iidx_v.at[p])

        def fire_in(dk, carry):
            d = dk >> 2
            p = dk & 3
            pltpu.async_copy(win_hbm.at[d].at[iidx_v.at[p]],
                             in_vals_v.at[dk], sem_in)
            return carry

        lax.fori_loop(0, DIM * NCHUNK, fire_in, 0)

        def drain_in(dk, carry):
            d = dk >> 2
            p = dk & 3
            pltpu.make_async_copy(win_hbm.at[d].at[iidx_v.at[p]],
                                  in_vals_v.at[dk], sem_in).wait()
            return carry

        for c in range(NCHUNK):
            for k in range(K):
                pltpu.sync_copy(oidxT_hbm.at[k, pl.ds(base + c * CS, CS)],
                                oidx_v.at[k])

            def fire_out(dk, carry):
                d = dk // K
                k = dk % K
                pltpu.async_copy(woutT_hbm.at[d].at[oidx_v.at[k]],
                                 vals_v.at[dk], sem_out)
                return carry

            lax.fori_loop(0, DIM * K, fire_out, 0)

            if c == 0:
                lax.fori_loop(0, DIM * NCHUNK, drain_in, 0)

            def drain_out(dk, carry):
                d = dk // K
                k = dk % K
                pltpu.make_async_copy(woutT_hbm.at[d].at[oidx_v.at[k]],
                                      vals_v.at[dk], sem_out).wait()
                return carry

            lax.fori_loop(0, DIM * K, drain_out, 0)

            def grp(g, carry, c=c):
                sbase = g * 16
                ivs = [in_vals_v[d * NCHUNK + c, pl.ds(sbase, 16)]
                       for d in range(DIM)]
                for k in range(K):
                    acc = vals_v[k, pl.ds(sbase, 16)] * ivs[0]
                    for d in range(1, DIM):
                        acc = acc + vals_v[d * K + k, pl.ds(sbase, 16)] * ivs[d]
                    out_v[k, pl.ds(c * CS + sbase, 16)] = acc
                return carry

            lax.fori_loop(0, CS // 16, grp, 0)

        for k in range(K):
            pltpu.sync_copy(out_v.at[k], out_hbm.at[k, pl.ds(base, SPT)])

    return sc_kernel


_SC_KERNEL = _build_kernel()


@jax.jit
def kernel(input_index_batch, output_indices_batch, W_in, W_out):
    idx_in1d = input_index_batch.astype(jnp.int32).reshape(B)
    oidxT = output_indices_batch.astype(jnp.int32).T
    woutT = W_out.T
    outT = _SC_KERNEL(idx_in1d, oidxT, W_in, woutT)
    return outT.T


# speedup vs baseline: 2.9522x; 2.9522x over previous
import functools

import jax
import jax.numpy as jnp
from jax import lax
from jax.experimental import pallas as pl
from jax.experimental.pallas import tpu as pltpu
from jax.experimental.pallas import tpu_sc as plsc

B = 16384
K = 21
DIM = 10
V = 1000000

NW = 32
SPT = B // NW
CS = 128
NCHUNK = SPT // CS


def _build_kernel():
    mesh = plsc.VectorSubcoreMesh(core_axis_name="c", subcore_axis_name="s")

    @functools.partial(
        pl.kernel,
        mesh=mesh,
        compiler_params=pltpu.CompilerParams(needs_layout_passes=False,
                                             use_tc_tiling_on_sc=False),
        out_type=jax.ShapeDtypeStruct((K, B), jnp.float32),
        scratch_types=[
            pltpu.VMEM((NCHUNK, CS), jnp.int32),
            pltpu.VMEM((DIM * NCHUNK, CS), jnp.float32),
            pltpu.VMEM((K, CS), jnp.int32),
            pltpu.VMEM((DIM * K, CS), jnp.float32),
            pltpu.VMEM((K, SPT), jnp.float32),
            pltpu.SemaphoreType.DMA,
            pltpu.SemaphoreType.DMA,
        ],
    )
    def sc_kernel(*refs):
        idx_in_hbm = refs[0]
        oidx_refs = refs[1:1 + K]
        win_refs = refs[1 + K:1 + K + DIM]
        wout_refs = refs[1 + K + DIM:1 + K + 2 * DIM]
        out_hbm = refs[1 + K + 2 * DIM]
        (iidx_v, in_vals_v, oidx_v, vals_v, out_v,
         sem_in, sem_out) = refs[2 + K + 2 * DIM:]

        wid = lax.axis_index("c") * 16 + lax.axis_index("s")
        base = wid * SPT

        for p in range(NCHUNK):
            pltpu.sync_copy(idx_in_hbm.at[pl.ds(base + p * CS, CS)],
                            iidx_v.at[p])

        for d in range(DIM):
            def fire_in(p, carry, d=d):
                pltpu.async_copy(win_refs[d].at[iidx_v.at[p]],
                                 in_vals_v.at[d * NCHUNK + p], sem_in)
                return carry

            lax.fori_loop(0, NCHUNK, fire_in, 0)

        def drain_in():
            for d in range(DIM):
                def drain1(p, carry, d=d):
                    pltpu.make_async_copy(win_refs[d].at[iidx_v.at[p]],
                                          in_vals_v.at[d * NCHUNK + p],
                                          sem_in).wait()
                    return carry

                lax.fori_loop(0, NCHUNK, drain1, 0)

        def chunk_body(c, carry):
            for k in range(K):
                pltpu.sync_copy(oidx_refs[k].at[pl.ds(base + c * CS, CS)],
                                oidx_v.at[k])

            for d in range(DIM):
                def fire_out(k, carry2, d=d):
                    pltpu.async_copy(wout_refs[d].at[oidx_v.at[k]],
                                     vals_v.at[d * K + k], sem_out)
                    return carry2

                lax.fori_loop(0, K, fire_out, 0)

            @pl.when(c == 0)
            def _():
                drain_in()

            for d in range(DIM):
                def drain_out(k, carry2, d=d):
                    pltpu.make_async_copy(wout_refs[d].at[oidx_v.at[k]],
                                          vals_v.at[d * K + k],
                                          sem_out).wait()
                    return carry2

                lax.fori_loop(0, K, drain_out, 0)

            def grp(g, carry2):
                sbase = g * 16
                ivs = [in_vals_v[d * NCHUNK + c, pl.ds(sbase, 16)]
                       for d in range(DIM)]
                for k in range(K):
                    acc = vals_v[k, pl.ds(sbase, 16)] * ivs[0]
                    for d in range(1, DIM):
                        acc = acc + vals_v[d * K + k, pl.ds(sbase, 16)] * ivs[d]
                    out_v[k, pl.ds(c * CS + sbase, 16)] = acc
                return carry2

            lax.fori_loop(0, CS // 16, grp, 0)
            return carry

        lax.fori_loop(0, NCHUNK, chunk_body, 0)

        for k in range(K):
            pltpu.sync_copy(out_v.at[k], out_hbm.at[k, pl.ds(base, SPT)])

    return sc_kernel


_SC_KERNEL = _build_kernel()


@jax.jit
def kernel(input_index_batch, output_indices_batch, W_in, W_out):
    iidx = input_index_batch.astype(jnp.int32).reshape(B)
    oidx = output_indices_batch.astype(jnp.int32)
    oidx_cols = [oidx[:, k] for k in range(K)]
    win_rows = [W_in[d] for d in range(DIM)]
    wout_cols = [W_out[:, d] for d in range(DIM)]
    outT = _SC_KERNEL(iidx, *oidx_cols, *win_rows, *wout_cols)
    return outT.T
